# baseline (speedup 1.0000x reference)
import functools

import jax
import jax.numpy as jnp
from jax import lax
from jax.experimental import pallas as pl
from jax.experimental.pallas import tpu as pltpu
from jax.experimental.pallas import tpu_sc as plsc

MASK_FRACTION = 0.75


def _rank_ids_body(keep, n_rows, nrow_ref, ncol_ref, rank_ref, mask_ref, ids_ref):
    L = nrow_ref.shape[2]
    b = nrow_ref[0]
    a = ncol_ref[...]
    ii = lax.broadcasted_iota(jnp.int32, (L, L), 0)
    jj = lax.broadcasted_iota(jnp.int32, (L, L), 1)
    contrib = (a < b) | ((a == b) & (ii < jj))
    rank_row = jnp.sum(contrib.astype(jnp.int32), axis=0, keepdims=True)
    rank_ref[0] = rank_row
    mask_ref[0] = (rank_row >= keep).astype(jnp.float32)
    kk = lax.broadcasted_iota(jnp.int32, (keep, L), 0)
    jl = lax.broadcasted_iota(jnp.int32, (keep, L), 1)
    n = pl.program_id(0)
    flat = jnp.where(rank_row == kk, jl + n * L, 0)
    ids_ref[...] = jnp.sum(flat, axis=1, keepdims=True)


def _rank_ids_call(noise, keep):
    N, L = noise.shape
    body = functools.partial(_rank_ids_body, keep, N)
    return pl.pallas_call(
        body,
        grid=(N,),
        in_specs=[
            pl.BlockSpec((1, 1, L), lambda n: (n, 0, 0)),
            pl.BlockSpec((L, 1), lambda n: (n, 0)),
        ],
        out_specs=[
            pl.BlockSpec((1, 1, L), lambda n: (n, 0, 0)),
            pl.BlockSpec((1, 1, L), lambda n: (n, 0, 0)),
            pl.BlockSpec((keep, 1), lambda n: (n, 0)),
        ],
        out_shape=[
            jax.ShapeDtypeStruct((N, 1, L), jnp.int32),
            jax.ShapeDtypeStruct((N, 1, L), jnp.float32),
            jax.ShapeDtypeStruct((N * keep, 1), jnp.int32),
        ],
    )(noise.reshape(N, 1, L), noise.reshape(N * L, 1))


def _sc_gather(ids_flat, x2):
    B = ids_flat.shape[0]
    D = x2.shape[1]
    info = plsc.get_sparse_core_info()
    nw = info.num_cores * info.num_subcores
    per_w = B // nw
    ch = 48
    nstep = per_w // ch
    mesh = plsc.VectorSubcoreMesh(core_axis_name="c", subcore_axis_name="s")

    @functools.partial(
        pl.kernel,
        mesh=mesh,
        out_type=jax.ShapeDtypeStruct((B, D), jnp.float32),
        scratch_types=[
            pltpu.VMEM((per_w,), jnp.int32),
            pltpu.VMEM((ch, D), jnp.float32),
            pltpu.VMEM((ch, D), jnp.float32),
            pltpu.SemaphoreType.DMA,
            pltpu.SemaphoreType.DMA,
        ],
    )
    def k(ids_hbm, x_hbm, out_hbm, ids_v, buf0, buf1, sem0, sem1):
        wid = lax.axis_index("s") * info.num_cores + lax.axis_index("c")
        base = wid * per_w
        pltpu.sync_copy(ids_hbm.at[pl.ds(base, per_w)], ids_v)
        bufs = (buf0, buf1)
        sems = (sem0, sem1)
        cps = []
        for s in range(nstep):
            cps.append(pltpu.async_copy(
                x_hbm.at[ids_v.at[pl.ds(s * ch, ch)]], bufs[s % 2], sems[s % 2]))
            if s >= 1:
                cps[s - 1].wait()
                pltpu.sync_copy(
                    bufs[(s - 1) % 2],
                    out_hbm.at[pl.ds(base + (s - 1) * ch, ch)])
        cps[nstep - 1].wait()
        pltpu.sync_copy(
            bufs[(nstep - 1) % 2],
            out_hbm.at[pl.ds(base + (nstep - 1) * ch, ch)])

    return k(ids_flat, x2)


def kernel(x):
    N, L, D = x.shape
    keep = int(L * (1 - MASK_FRACTION))
    noise = jax.random.uniform(jax.random.key(42), (N, L), dtype=jnp.float32)
    rank3, mask3, ids_col = _rank_ids_call(noise, keep)
    x_masked = _sc_gather(ids_col.reshape(N * keep), x.reshape(N * L, D))
    return (x_masked.reshape(N, keep, D),
            mask3.reshape(N, L),
            rank3.reshape(N, L))


# speedup vs baseline: 4.8915x; 4.8915x over previous
import functools

import jax
import jax.numpy as jnp
from jax import lax
from jax.experimental import pallas as pl
from jax.experimental.pallas import tpu as pltpu
from jax.experimental.pallas import tpu_sc as plsc

MASK_FRACTION = 0.75
_V = 16


def _sc_mask_and_gather(noise_flat, x2, N, L, keep):
    D = x2.shape[1]
    info = plsc.get_sparse_core_info()
    nw = info.num_cores * info.num_subcores
    rows_w = N // nw
    nv = L // _V
    kv = keep // _V
    ch = 72
    nch = rows_w * keep // ch
    mesh = plsc.VectorSubcoreMesh(core_axis_name="c", subcore_axis_name="s")

    @functools.partial(
        pl.kernel,
        mesh=mesh,
        compiler_params=pltpu.CompilerParams(needs_layout_passes=False),
        out_type=(
            jax.ShapeDtypeStruct((N * keep, D), jnp.float32),
            jax.ShapeDtypeStruct((N * L,), jnp.float32),
            jax.ShapeDtypeStruct((N * L,), jnp.int32),
        ),
        scratch_types=[
            pltpu.VMEM((rows_w * L,), jnp.float32),
            pltpu.VMEM((L,), jnp.int32),
            pltpu.VMEM((L,), jnp.int32),
            pltpu.VMEM((L,), jnp.int32),
            pltpu.VMEM((L,), jnp.int32),
            pltpu.VMEM((256,), jnp.int32),
            pltpu.VMEM((256,), jnp.int32),
            pltpu.VMEM((rows_w * keep,), jnp.int32),
            pltpu.VMEM((rows_w * L,), jnp.float32),
            pltpu.VMEM((rows_w * L,), jnp.int32),
            pltpu.VMEM((ch, D), jnp.float32),
            pltpu.VMEM((ch, D), jnp.float32),
            pltpu.SemaphoreType.DMA,
            pltpu.SemaphoreType.DMA,
            pltpu.SemaphoreType.DMA,
            pltpu.SemaphoreType.DMA,
        ],
    )
    def k(noise_hbm, x_hbm, xm_hbm, mask_hbm, rest_hbm,
          noise_v, ka, va, kb, vb, hist_v, off_v, ids_v, mask4_v, rest4_v,
          buf0, buf1, g0, g1, o0, o1):
        wid = lax.axis_index("s") * info.num_cores + lax.axis_index("c")
        row0 = wid * rows_w
        pltpu.sync_copy(noise_hbm.at[pl.ds(row0 * L, rows_w * L)], noise_v)

        def row_body(r, carry):
            for c in range(nv):
                nvv = noise_v[pl.ds(r * L + c * _V, _V)]
                ka[pl.ds(c * _V, _V)] = (nvv * 8388608.0).astype(jnp.int32)
                va[pl.ds(c * _V, _V)] = lax.iota(jnp.int32, _V) + c * _V
            pp = ((ka, va, kb, vb), (kb, vb, ka, va), (ka, va, kb, vb))
            for p, shift in enumerate((0, 8, 16)):
                ksrc, vsrc, kdst, vdst = pp[p]
                for h in range(256 // _V):
                    hist_v[pl.ds(h * _V, _V)] = jnp.zeros((_V,), jnp.int32)
                for c in range(nv):
                    d = (ksrc[pl.ds(c * _V, _V)] >> shift) & 255
                    cnt, last = plsc.scan_count(d)
                    plsc.addupdate_scatter(hist_v, [d], cnt, mask=last)
                carry_s = jnp.int32(0)
                for h in range(256 // _V):
                    hv = hist_v[pl.ds(h * _V, _V)]
                    inc = plsc.cumsum(hv)
                    off_v[pl.ds(h * _V, _V)] = inc - hv + carry_s
                    carry_s = carry_s + jnp.sum(hv)
                for c in range(nv):
                    kk = ksrc[pl.ds(c * _V, _V)]
                    vv = vsrc[pl.ds(c * _V, _V)]
                    d = (kk >> shift) & 255
                    cnt, last = plsc.scan_count(d)
                    base = plsc.load_gather(off_v, [d])
                    pos = base + cnt - 1
                    plsc.store_scatter(kdst, [pos], kk)
                    plsc.store_scatter(vdst, [pos], vv)
                    plsc.addupdate_scatter(off_v, [d], cnt, mask=last)
            for q in range(nv):
                sv = vb[pl.ds(q * _V, _V)]
                kidx = lax.iota(jnp.int32, _V) + q * _V
                plsc.store_scatter(rest4_v, [sv + r * L], kidx)
                plsc.store_scatter(
                    mask4_v, [sv + r * L],
                    jnp.where(kidx >= keep, 1.0, 0.0).astype(jnp.float32))
            for q in range(kv):
                sv = vb[pl.ds(q * _V, _V)]
                ids_v[pl.ds(r * keep + q * _V, _V)] = sv + (row0 + r) * L
            return carry

        lax.fori_loop(0, rows_w, row_body, 0)

        pltpu.sync_copy(mask4_v, mask_hbm.at[pl.ds(row0 * L, rows_w * L)])
        pltpu.sync_copy(rest4_v, rest_hbm.at[pl.ds(row0 * L, rows_w * L)])

        bufs = (buf0, buf1)
        gsems = (g0, g1)
        osems = (o0, o1)
        obase = row0 * keep
        gcp = [None] * nch
        ocp = [None] * nch
        for c in range(nch):
            if c >= 2:
                ocp[c - 2].wait()
            gcp[c] = pltpu.async_copy(
                x_hbm.at[ids_v.at[pl.ds(c * ch, ch)]], bufs[c % 2], gsems[c % 2])
            if c >= 1:
                gcp[c - 1].wait()
                ocp[c - 1] = pltpu.async_copy(
                    bufs[(c - 1) % 2],
                    xm_hbm.at[pl.ds(obase + (c - 1) * ch, ch)],
                    osems[(c - 1) % 2])
        gcp[nch - 1].wait()
        ocp[nch - 1] = pltpu.async_copy(
            bufs[(nch - 1) % 2],
            xm_hbm.at[pl.ds(obase + (nch - 1) * ch, ch)],
            osems[(nch - 1) % 2])
        ocp[nch - 2].wait()
        ocp[nch - 1].wait()

    return k(noise_flat, x2)


def kernel(x):
    N, L, D = x.shape
    keep = int(L * (1 - MASK_FRACTION))
    noise = jax.random.uniform(jax.random.key(42), (N, L), dtype=jnp.float32)
    xm, mask, rest = _sc_mask_and_gather(
        noise.reshape(N * L), x.reshape(N * L, D), N, L, keep)
    return (xm.reshape(N, keep, D),
            mask.reshape(N, L),
            rest.reshape(N, L))
